# baseline (speedup 1.0000x reference)
import functools

import jax
import jax.numpy as jnp
from jax import lax
from jax.experimental import pallas as pl
from jax.experimental.pallas import tpu as pltpu
from jax.experimental.pallas import tpu_sc as plsc

NUM_ITEMS = 1000000
D = 32
B = 16384
NC = 2
NS = 16
L = 16
HITEMS = B // NC
ICH = 128
NCHUNK = HITEMS // ICH
SIG = HITEMS // NS

_mesh = plsc.VectorSubcoreMesh(
    core_axis_name="c", subcore_axis_name="s", num_cores=NC, num_subcores=NS
)


@functools.partial(
    pl.kernel,
    out_type=[
        jax.ShapeDtypeStruct((B,), jnp.float32),
        jax.ShapeDtypeStruct((D, B), jnp.float32),
        jax.ShapeDtypeStruct((D, B), jnp.float32),
    ],
    mesh=_mesh,
    compiler_params=pltpu.CompilerParams(
        use_tc_tiling_on_sc=False, needs_layout_passes=False
    ),
    scratch_types=[
        pltpu.VMEM((HITEMS,), jnp.int32),
        pltpu.VMEM((HITEMS,), jnp.float32),
        pltpu.VMEM((HITEMS,), jnp.float32),
        pltpu.VMEM((HITEMS,), jnp.float32),
        pltpu.VMEM((HITEMS,), jnp.float32),
        pltpu.VMEM((HITEMS,), jnp.float32),
        pltpu.VMEM((NS, SIG), jnp.float32),
        pltpu.VMEM((SIG,), jnp.float32),
        pltpu.VMEM((D,), jnp.float32),
        pltpu.VMEM((L,), jnp.float32),
        pltpu.VMEM_SHARED((NS, HITEMS), jnp.float32),
        pltpu.SemaphoreType.DMA,
        pltpu.SemaphoreType.DMA,
    ],
)
def _fedrap_sc(idx_hbm, pt_hbm, ct_hbm, w_hbm, b_hbm,
               rating_hbm, outp_hbm, outc_hbm,
               idx_v, pa_v, pb_v, ca_v, cb_v, t_v, red_v, sg_v, w_v, b_v,
               acc_sh, gsem, osem):
    c = lax.axis_index("c")
    s = lax.axis_index("s")
    jlo = s
    jhi = s + NS
    hbase = c * HITEMS

    pltpu.sync_copy(idx_hbm.at[pl.ds(hbase, HITEMS)], idx_v)
    pltpu.sync_copy(w_hbm, w_v)
    pltpu.sync_copy(b_hbm, b_v)

    srcs = (
        (pt_hbm.at[jlo], pa_v),
        (pt_hbm.at[jhi], pb_v),
        (ct_hbm.at[jlo], ca_v),
        (ct_hbm.at[jhi], cb_v),
    )

    def fire(ch, carry):
        isl = idx_v.at[pl.ds(ch * ICH, ICH)]
        dsl = pl.ds(ch * ICH, ICH)
        for src, dst in srcs:
            pltpu.async_copy(src.at[isl], dst.at[dsl], gsem)
        return carry

    def drain(ch, carry):
        isl = idx_v.at[pl.ds(ch * ICH, ICH)]
        dsl = pl.ds(ch * ICH, ICH)
        for src, dst in srcs:
            pltpu.make_async_copy(src.at[isl], dst.at[dsl], gsem).wait()
        return carry

    lax.fori_loop(0, NCHUNK, fire, 0, unroll=False)
    lax.fori_loop(0, NCHUNK, drain, 0, unroll=False)

    out_cps = [
        pltpu.async_copy(pa_v, outp_hbm.at[jlo, pl.ds(hbase, HITEMS)], osem),
        pltpu.async_copy(pb_v, outp_hbm.at[jhi, pl.ds(hbase, HITEMS)], osem),
        pltpu.async_copy(ca_v, outc_hbm.at[jlo, pl.ds(hbase, HITEMS)], osem),
        pltpu.async_copy(cb_v, outc_hbm.at[jhi, pl.ds(hbase, HITEMS)], osem),
    ]

    lane = lax.iota(jnp.int32, L)
    w_lo = w_v[pl.ds(0, L)]
    w_hi = w_v[pl.ds(L, L)]
    wa = jnp.sum(jnp.where(lane == s, w_lo, 0.0))
    wb = jnp.sum(jnp.where(lane == s, w_hi, 0.0))

    def part(g, carry):
        sl = pl.ds(g * L, L)
        t_v[sl] = (pa_v[sl] + ca_v[sl]) * wa + (pb_v[sl] + cb_v[sl]) * wb
        return carry

    lax.fori_loop(0, HITEMS // L, part, 0, unroll=False)

    pltpu.sync_copy(t_v, acc_sh.at[s])
    plsc.subcore_barrier()
    pltpu.sync_copy(acc_sh.at[:, pl.ds(s * SIG, SIG)], red_v)
    bias = b_v[...]

    def sig(g, carry):
        sl = pl.ds(g * L, L)
        acc = red_v[0, sl]
        for k in range(1, NS):
            acc = acc + red_v[k, sl]
        sg_v[sl] = 1.0 / (1.0 + jnp.exp(-(acc + bias)))
        return carry

    lax.fori_loop(0, SIG // L, sig, 0, unroll=False)
    pltpu.sync_copy(sg_v, rating_hbm.at[pl.ds(hbase + s * SIG, SIG)])

    for cp in out_cps:
        cp.wait()


def kernel(item_indices, personality, commonality, W, b):
    idx = item_indices.astype(jnp.int32)
    w_flat = W.reshape(D).astype(jnp.float32)
    b_splat = jnp.broadcast_to(b.astype(jnp.float32), (L,))
    rating, item_pt, item_ct = _fedrap_sc(
        idx, personality.T, commonality.T, w_flat, b_splat
    )
    return (rating.reshape(B, 1), item_pt.T, item_ct.T)


# speedup vs baseline: 5.4783x; 5.4783x over previous
import functools

import jax
import jax.numpy as jnp
from jax import lax
from jax.experimental import pallas as pl
from jax.experimental.pallas import tpu as pltpu
from jax.experimental.pallas import tpu_sc as plsc

NUM_ITEMS = 1000000
D = 32
B = 16384
NC = 2
NS = 16
L = 16
NW = NC * NS
BPW = B // NW
ICH = 128
NCHUNK = BPW // ICH
GROUPS = BPW // L

_mesh = plsc.VectorSubcoreMesh(
    core_axis_name="c", subcore_axis_name="s", num_cores=NC, num_subcores=NS
)


@functools.partial(
    pl.kernel,
    out_type=[
        jax.ShapeDtypeStruct((B,), jnp.float32),
        jax.ShapeDtypeStruct((B, D), jnp.float32),
        jax.ShapeDtypeStruct((B, D), jnp.float32),
    ],
    mesh=_mesh,
    compiler_params=pltpu.CompilerParams(
        use_tc_tiling_on_sc=False, needs_layout_passes=False
    ),
    scratch_types=[
        pltpu.VMEM((NCHUNK, ICH), jnp.int32),
        pltpu.VMEM((BPW, D), jnp.float32),
        pltpu.VMEM((BPW, D), jnp.float32),
        pltpu.VMEM((BPW,), jnp.float32),
        pltpu.VMEM((D,), jnp.float32),
        pltpu.VMEM((L,), jnp.float32),
        pltpu.SemaphoreType.DMA,
        pltpu.SemaphoreType.DMA,
    ],
)
def _fedrap_sc(idx_hbm, p_hbm, c_hbm, w_hbm, b_hbm,
               rating_hbm, outp_hbm, outc_hbm,
               idx_v, p_v, c_v, r_v, w_v, b_v, gsem, osem):
    wid = lax.axis_index("s") * NC + lax.axis_index("c")
    base = wid * BPW

    pltpu.sync_copy(idx_hbm.at[pl.ds(wid * NCHUNK, NCHUNK)], idx_v)
    pltpu.sync_copy(w_hbm, w_v)
    pltpu.sync_copy(b_hbm, b_v)

    for i in range(NCHUNK):
        pltpu.async_copy(
            p_hbm.at[idx_v.at[i]], p_v.at[pl.ds(i * ICH, ICH)], gsem
        )
        pltpu.async_copy(
            c_hbm.at[idx_v.at[i]], c_v.at[pl.ds(i * ICH, ICH)], gsem
        )
    for i in range(NCHUNK):
        pltpu.make_async_copy(
            p_hbm.at[idx_v.at[i]], p_v.at[pl.ds(i * ICH, ICH)], gsem
        ).wait()
        pltpu.make_async_copy(
            c_hbm.at[idx_v.at[i]], c_v.at[pl.ds(i * ICH, ICH)], gsem
        ).wait()

    op = pltpu.async_copy(p_v, outp_hbm.at[pl.ds(base, BPW)], osem)
    oc = pltpu.async_copy(c_v, outc_hbm.at[pl.ds(base, BPW)], osem)

    bias = b_v[...]
    lane = lax.iota(jnp.int32, L)
    w_lo = w_v[pl.ds(0, L)]
    w_hi = w_v[pl.ds(L, L)]

    def group(g, carry):
        rows = g * L + lane
        acc = bias
        for j in range(D):
            col = jnp.full((L,), j, jnp.int32)
            pv = plsc.load_gather(p_v, [rows, col])
            cv = plsc.load_gather(c_v, [rows, col])
            w_j = w_lo[j] if j < L else w_hi[j - L]
            acc = acc + (pv + cv) * w_j
        r_v[pl.ds(g * L, L)] = 1.0 / (1.0 + jnp.exp(-acc))
        return carry

    lax.fori_loop(0, GROUPS, group, 0, unroll=False)

    pltpu.sync_copy(r_v, rating_hbm.at[pl.ds(base, BPW)])
    op.wait()
    oc.wait()


def kernel(item_indices, personality, commonality, W, b):
    idx = item_indices.astype(jnp.int32).reshape(NW * NCHUNK, ICH)
    w_flat = W.reshape(D).astype(jnp.float32)
    b_splat = jnp.broadcast_to(b.astype(jnp.float32), (L,))
    rating, item_p, item_c = _fedrap_sc(
        idx, personality, commonality, w_flat, b_splat
    )
    return (rating.reshape(B, 1), item_p, item_c)
